# baseline (speedup 1.0000x reference)
import functools

import jax
import jax.numpy as jnp
from jax import lax
from jax.experimental import pallas as pl
from jax.experimental.pallas import tpu as pltpu
from jax.experimental.pallas import tpu_sc as plsc

_NC = 2
_NS = 16
_NW = _NC * _NS
_B = 64

_AF = 39
_AFP = 64
_H = 128
_EP = 163840
_AP = 12288


@functools.lru_cache(maxsize=None)
def _make_gsum(nb, k, d, bf=False):
    bpw = nb // (_NW * _B)
    ns = 4 if bpw % 4 == 0 else 2
    assert nb == bpw * _NW * _B and bpw % ns == 0 and bpw >= ns
    mesh = plsc.VectorSubcoreMesh(
        core_axis_name="c", subcore_axis_name="s",
        num_cores=_NC, num_subcores=_NS)

    def body(table_hbm, idxt_hbm, out_hbm, idx_v, buf_v, sem_i, sem_g, sem_o):
        wid = lax.axis_index("s") * _NC + lax.axis_index("c")
        first = wid * bpw

        def start_idx(b, s):
            base = (first + b) * _B
            for j in range(k):
                pltpu.async_copy(idxt_hbm.at[pl.ds(j * nb + base, _B)],
                                 idx_v.at[s, j], sem_i)

        def wait_idx(s):
            for j in range(k):
                pltpu.make_async_copy(idxt_hbm.at[pl.ds(0, _B)],
                                      idx_v.at[s, j], sem_i).wait()

        def fire_gathers(s):
            for j in range(k):
                pltpu.async_copy(table_hbm.at[idx_v.at[s, j]],
                                 buf_v.at[s, j], sem_g)

        def drain_gathers(s):
            for j in range(k):
                pltpu.make_async_copy(table_hbm.at[idx_v.at[s, j]],
                                      buf_v.at[s, j], sem_g).wait()

        def start_out(b, s):
            base = (first + b) * _B
            pltpu.async_copy(buf_v.at[s, 0], out_hbm.at[pl.ds(base, _B)],
                             sem_o)

        def wait_out(s):
            pltpu.make_async_copy(buf_v.at[s, 0], out_hbm.at[pl.ds(0, _B)],
                                  sem_o).wait()

        def reduce_block(s):
            if k == 1:
                return

            def row(r, c2):
                if bf:
                    fmt = plsc.PackFormat.INTERLEAVED
                    for j in range(d // 32):
                        sl = pl.ds(j * 32, 32)
                        acc_a, acc_b = plsc.unpack(buf_v[s, 0, r, sl],
                                                   format=fmt)
                        for q in range(1, k):
                            qa, qb = plsc.unpack(buf_v[s, q, r, sl],
                                                 format=fmt)
                            acc_a = acc_a + qa
                            acc_b = acc_b + qb
                        buf_v[s, 0, r, sl] = plsc.pack(acc_a, acc_b,
                                                       format=fmt)
                else:
                    for j in range(d // 16):
                        sl = pl.ds(j * 16, 16)
                        acc = buf_v[s, 0, r, sl]
                        for q in range(1, k):
                            acc = acc + buf_v[s, q, r, sl]
                        buf_v[s, 0, r, sl] = acc
                return c2

            lax.fori_loop(0, _B, row, 0)

        def step(b, s):
            drain_gathers(s)

            @pl.when(b + ns - 1 < bpw)
            def _():
                sg = (s + ns - 1) % ns
                wait_idx(sg)

                @pl.when(b >= 1)
                def _():
                    wait_out(sg)

                fire_gathers(sg)

                @pl.when(b + ns < bpw)
                def _():
                    start_idx(b + ns, s)

            reduce_block(s)
            start_out(b, s)

        for j in range(ns - 1):
            start_idx(j, j)
            wait_idx(j)
            fire_gathers(j)
        start_idx(ns - 1, ns - 1)

        def rotation(i, carry):
            for j in range(ns):
                step(ns * i + j, j)
            return carry

        lax.fori_loop(0, bpw // ns, rotation, 0)
        for j in range(ns):
            wait_out(j)

    dt = jnp.bfloat16 if bf else jnp.float32
    return pl.kernel(
        body,
        out_type=jax.ShapeDtypeStruct((nb, d), dt),
        mesh=mesh,
        compiler_params=pltpu.CompilerParams(
            use_tc_tiling_on_sc=False,
            needs_layout_passes=not bf),
        scratch_types=[
            pltpu.VMEM((ns, k, _B), jnp.int32),
            pltpu.VMEM((ns, k, _B, d), dt),
            pltpu.SemaphoreType.DMA,
            pltpu.SemaphoreType.DMA,
            pltpu.SemaphoreType.DMA,
        ],
    )


def _gsum(table, idxt):
    k, nb = idxt.shape
    bf = table.dtype == jnp.bfloat16
    return _make_gsum(nb, k, table.shape[1], bf)(table, idxt.reshape(-1))


def _dot_t(x, w):
    return lax.dot_general(x, w, (((1,), (1,)), ((), ())),
                           preferred_element_type=jnp.float32)


def _prologue_body(fb_ref, ms_ref, we_ref, w2_ref, h0_ref, c_ref):
    h0 = jnp.maximum(_dot_t(fb_ref[...], we_ref[...]), 0.0)
    h0_ref[...] = h0.astype(jnp.bfloat16)
    c_ref[...] = h0 + _dot_t(ms_ref[...], w2_ref[...])


def _iter_body(s_ref, c_ref, w1_ref, o_ref):
    h = jnp.maximum(c_ref[...] + _dot_t(s_ref[...], w1_ref[...]), 0.0)
    o_ref[...] = h.astype(jnp.bfloat16)


def _final_body(sa_ref, ms_ref, wo1_ref, wo2_ref, o_ref):
    acc = _dot_t(sa_ref[...], wo1_ref[...]) + _dot_t(ms_ref[...], wo2_ref[...])
    o_ref[...] = jnp.maximum(acc, 0.0)


def _full(shape):
    return pl.BlockSpec(shape, lambda i: (0, 0))


def _rows(bm, d):
    return pl.BlockSpec((bm, d), lambda i: (i, 0))


def kernel(fatoms, fbonds, aoutgraph, bgraph, aingraph, scope, all_bonds,
           W_ein, W_edge, W_eout):
    f32 = jnp.float32
    n_b, bf = fbonds.shape
    n_a = fatoms.shape[0]

    i32 = jnp.int32
    bf16 = jnp.bfloat16
    fat_z = jnp.pad(jnp.concatenate(
        [jnp.zeros((1, _AF), f32), fatoms], axis=0),
        ((0, 7), (0, _AFP - _AF))).astype(bf16)
    sidx = jnp.concatenate([
        jnp.zeros((1,), i32),
        all_bonds[1:, 0].astype(i32) + 1,
        jnp.zeros((_EP - n_b,), i32),
    ]).reshape(1, _EP)
    bgt = jnp.pad(bgraph.astype(i32), ((0, _EP - n_b), (0, 0))).T
    agt = jnp.pad(aingraph.astype(i32), ((0, _AP - n_a), (0, 0))).T

    fb16 = jnp.pad(fbonds, ((0, 0), (0, 16 - bf)))
    we16 = jnp.pad(W_ein, ((0, 0), (0, 16 - bf)))
    w1b = W_edge[:, :_H].astype(bf16)
    w2b = jnp.pad(W_edge[:, _H:], ((0, 0), (0, _AFP - _AF))).astype(bf16)
    wo1b = W_eout[:, :_H].astype(bf16)
    wo2b = jnp.pad(W_eout[:, _H:], ((0, 0), (0, _AFP - _AF))).astype(bf16)

    msg = _gsum(fat_z, sidx)
    msum_e = _gsum(msg, bgt)
    msum_a = _gsum(msg, agt)

    bm = 640
    grid_e = n_b // bm
    h0, c = pl.pallas_call(
        _prologue_body,
        grid=(grid_e,),
        in_specs=[_rows(bm, 16), _rows(bm, _AFP),
                  _full((_H, 16)), _full((_H, _AFP))],
        out_specs=[_rows(bm, _H), _rows(bm, _H)],
        out_shape=[jax.ShapeDtypeStruct((n_b, _H), bf16),
                   jax.ShapeDtypeStruct((n_b, _H), f32)],
    )(fb16, msum_e, we16, w2b)

    h_e = h0
    for _ in range(3):
        s = _gsum(h_e, bgt)
        h_e = pl.pallas_call(
            _iter_body,
            grid=(grid_e,),
            in_specs=[_rows(bm, _H), _rows(bm, _H), _full((_H, _H))],
            out_specs=_rows(bm, _H),
            out_shape=jax.ShapeDtypeStruct((n_b, _H), bf16),
        )(s, c, w1b)

    s_a = _gsum(h_e, agt)
    bma = 512
    out_p = pl.pallas_call(
        _final_body,
        grid=(_AP // bma,),
        in_specs=[_rows(bma, _H), _rows(bma, _AFP),
                  _full((_H, _H)), _full((_H, _AFP))],
        out_specs=_rows(bma, _H),
        out_shape=jax.ShapeDtypeStruct((_AP, _H), f32),
    )(s_a, msum_a, wo1b, wo2b)

    return out_p[:n_a].T


# speedup vs baseline: 1.2017x; 1.2017x over previous
import functools

import jax
import jax.numpy as jnp
from jax import lax
from jax.experimental import pallas as pl
from jax.experimental.pallas import tpu as pltpu
from jax.experimental.pallas import tpu_sc as plsc

_NC = 2
_NS = 16
_NW = _NC * _NS
_B = 64

_AF = 39
_AFP = 64
_H = 128
_EP = 163840
_AP = 12288


@functools.lru_cache(maxsize=None)
def _make_gsum(nb, k, d, bf=False):
    bpw = nb // (_NW * _B)
    ns = 4 if bpw % 4 == 0 else 2
    assert nb == bpw * _NW * _B and bpw % ns == 0 and bpw >= ns
    mesh = plsc.VectorSubcoreMesh(
        core_axis_name="c", subcore_axis_name="s",
        num_cores=_NC, num_subcores=_NS)

    def body(table_hbm, idxt_hbm, out_hbm, idx_v, buf_v, sem_i, sem_g, sem_o):
        wid = lax.axis_index("s") * _NC + lax.axis_index("c")

        def start_idx(b, s):
            base = (b * _NW + wid) * _B
            for j in range(k):
                pltpu.async_copy(idxt_hbm.at[pl.ds(j * nb + base, _B)],
                                 idx_v.at[s, j], sem_i)

        def wait_idx(s):
            for j in range(k):
                pltpu.make_async_copy(idxt_hbm.at[pl.ds(0, _B)],
                                      idx_v.at[s, j], sem_i).wait()

        def fire_gathers(s):
            for j in range(k):
                pltpu.async_copy(table_hbm.at[idx_v.at[s, j]],
                                 buf_v.at[s, j], sem_g)

        def drain_gathers(s):
            for j in range(k):
                pltpu.make_async_copy(table_hbm.at[idx_v.at[s, j]],
                                      buf_v.at[s, j], sem_g).wait()

        def start_out(b, s):
            base = (b * _NW + wid) * _B
            pltpu.async_copy(buf_v.at[s, 0], out_hbm.at[pl.ds(base, _B)],
                             sem_o)

        def wait_out(s):
            pltpu.make_async_copy(buf_v.at[s, 0], out_hbm.at[pl.ds(0, _B)],
                                  sem_o).wait()

        def reduce_block(s):
            if k == 1:
                return

            def row(r, c2):
                if bf:
                    fmt = plsc.PackFormat.INTERLEAVED
                    for j in range(d // 32):
                        sl = pl.ds(j * 32, 32)
                        acc_a, acc_b = plsc.unpack(buf_v[s, 0, r, sl],
                                                   format=fmt)
                        for q in range(1, k):
                            qa, qb = plsc.unpack(buf_v[s, q, r, sl],
                                                 format=fmt)
                            acc_a = acc_a + qa
                            acc_b = acc_b + qb
                        buf_v[s, 0, r, sl] = plsc.pack(acc_a, acc_b,
                                                       format=fmt)
                else:
                    for j in range(d // 16):
                        sl = pl.ds(j * 16, 16)
                        acc = buf_v[s, 0, r, sl]
                        for q in range(1, k):
                            acc = acc + buf_v[s, q, r, sl]
                        buf_v[s, 0, r, sl] = acc
                return c2

            lax.fori_loop(0, _B, row, 0)

        def step(b, s):
            drain_gathers(s)

            @pl.when(b + ns - 1 < bpw)
            def _():
                sg = (s + ns - 1) % ns
                wait_idx(sg)

                @pl.when(b >= 1)
                def _():
                    wait_out(sg)

                fire_gathers(sg)

                @pl.when(b + ns < bpw)
                def _():
                    start_idx(b + ns, s)

            reduce_block(s)
            start_out(b, s)

        for j in range(ns - 1):
            start_idx(j, j)
            wait_idx(j)
            fire_gathers(j)
        start_idx(ns - 1, ns - 1)

        def rotation(i, carry):
            for j in range(ns):
                step(ns * i + j, j)
            return carry

        lax.fori_loop(0, bpw // ns, rotation, 0)
        for j in range(ns):
            wait_out(j)

    dt = jnp.bfloat16 if bf else jnp.float32
    return pl.kernel(
        body,
        out_type=jax.ShapeDtypeStruct((nb, d), dt),
        mesh=mesh,
        compiler_params=pltpu.CompilerParams(
            use_tc_tiling_on_sc=False,
            needs_layout_passes=not bf),
        scratch_types=[
            pltpu.VMEM((ns, k, _B), jnp.int32),
            pltpu.VMEM((ns, k, _B, d), dt),
            pltpu.SemaphoreType.DMA,
            pltpu.SemaphoreType.DMA,
            pltpu.SemaphoreType.DMA,
        ],
    )


def _gsum(table, idxt):
    k, nb = idxt.shape
    bf = table.dtype == jnp.bfloat16
    return _make_gsum(nb, k, table.shape[1], bf)(table, idxt.reshape(-1))


def _dot_t(x, w):
    return lax.dot_general(x, w, (((1,), (1,)), ((), ())),
                           preferred_element_type=jnp.float32)


def _prologue_body(fb_ref, ms_ref, we_ref, w2_ref, h0_ref, c_ref):
    h0 = jnp.maximum(_dot_t(fb_ref[...], we_ref[...]), 0.0)
    h0_ref[...] = h0.astype(jnp.bfloat16)
    c_ref[...] = h0 + _dot_t(ms_ref[...], w2_ref[...])


def _iter_body(s_ref, c_ref, w1_ref, o_ref):
    h = jnp.maximum(c_ref[...] + _dot_t(s_ref[...], w1_ref[...]), 0.0)
    o_ref[...] = h.astype(jnp.bfloat16)


def _final_body(sa_ref, ms_ref, wo1_ref, wo2_ref, o_ref):
    acc = _dot_t(sa_ref[...], wo1_ref[...]) + _dot_t(ms_ref[...], wo2_ref[...])
    o_ref[...] = jnp.maximum(acc, 0.0)


def _full(shape):
    return pl.BlockSpec(shape, lambda i: (0, 0))


def _rows(bm, d):
    return pl.BlockSpec((bm, d), lambda i: (i, 0))


def kernel(fatoms, fbonds, aoutgraph, bgraph, aingraph, scope, all_bonds,
           W_ein, W_edge, W_eout):
    f32 = jnp.float32
    n_b, bf = fbonds.shape
    n_a = fatoms.shape[0]

    i32 = jnp.int32
    bf16 = jnp.bfloat16
    fat_z = jnp.pad(jnp.concatenate(
        [jnp.zeros((1, _AF), f32), fatoms], axis=0),
        ((0, 7), (0, _AFP - _AF))).astype(bf16)
    sidx = jnp.concatenate([
        jnp.zeros((1,), i32),
        all_bonds[1:, 0].astype(i32) + 1,
        jnp.zeros((_EP - n_b,), i32),
    ]).reshape(1, _EP)
    bgt = jnp.pad(bgraph.astype(i32), ((0, _EP - n_b), (0, 0))).T
    agt = jnp.pad(aingraph.astype(i32), ((0, _AP - n_a), (0, 0))).T

    fb16 = jnp.pad(fbonds, ((0, 0), (0, 16 - bf)))
    we16 = jnp.pad(W_ein, ((0, 0), (0, 16 - bf)))
    w1b = W_edge[:, :_H].astype(bf16)
    w2b = jnp.pad(W_edge[:, _H:], ((0, 0), (0, _AFP - _AF))).astype(bf16)
    wo1b = W_eout[:, :_H].astype(bf16)
    wo2b = jnp.pad(W_eout[:, _H:], ((0, 0), (0, _AFP - _AF))).astype(bf16)

    msg = _gsum(fat_z, sidx)
    msum_e = _gsum(msg, bgt)
    msum_a = _gsum(msg, agt)

    bm = 640
    grid_e = n_b // bm
    h0, c = pl.pallas_call(
        _prologue_body,
        grid=(grid_e,),
        in_specs=[_rows(bm, 16), _rows(bm, _AFP),
                  _full((_H, 16)), _full((_H, _AFP))],
        out_specs=[_rows(bm, _H), _rows(bm, _H)],
        out_shape=[jax.ShapeDtypeStruct((n_b, _H), bf16),
                   jax.ShapeDtypeStruct((n_b, _H), f32)],
    )(fb16, msum_e, we16, w2b)

    h_e = h0
    for _ in range(3):
        s = _gsum(h_e, bgt)
        h_e = pl.pallas_call(
            _iter_body,
            grid=(grid_e,),
            in_specs=[_rows(bm, _H), _rows(bm, _H), _full((_H, _H))],
            out_specs=_rows(bm, _H),
            out_shape=jax.ShapeDtypeStruct((n_b, _H), bf16),
        )(s, c, w1b)

    s_a = _gsum(h_e, agt)
    bma = 512
    out_p = pl.pallas_call(
        _final_body,
        grid=(_AP // bma,),
        in_specs=[_rows(bma, _H), _rows(bma, _AFP),
                  _full((_H, _H)), _full((_H, _AFP))],
        out_specs=_rows(bma, _H),
        out_shape=jax.ShapeDtypeStruct((_AP, _H), f32),
    )(s_a, msum_a, wo1b, wo2b)

    return out_p[:n_a].T


# speedup vs baseline: 1.2056x; 1.0032x over previous
import functools

import jax
import jax.numpy as jnp
from jax import lax
from jax.experimental import pallas as pl
from jax.experimental.pallas import tpu as pltpu
from jax.experimental.pallas import tpu_sc as plsc

_NC = 2
_NS = 16
_NW = _NC * _NS
_B = 64

_AF = 39
_AFP = 64
_H = 128
_EP = 163840
_AP = 12288


@functools.lru_cache(maxsize=None)
def _make_gsum(nb, k, d, bf=False):
    size = 2 if bf else 4
    blk = _B
    if nb % (_NW * 128 * 2) == 0 and 2 * k * 128 * d * size <= 400_000:
        blk = 128
    bpw = nb // (_NW * blk)
    ns = 2
    if bpw % 4 == 0 and 4 * k * blk * d * size <= 400_000:
        ns = 4
    assert nb == bpw * _NW * blk and bpw % ns == 0 and bpw >= ns
    mesh = plsc.VectorSubcoreMesh(
        core_axis_name="c", subcore_axis_name="s",
        num_cores=_NC, num_subcores=_NS)

    def body(table_hbm, idxt_hbm, out_hbm, idx_v, buf_v, sem_i, sem_g, sem_o):
        wid = lax.axis_index("s") * _NC + lax.axis_index("c")

        def start_idx(b, s):
            base = (b * _NW + wid) * blk
            for j in range(k):
                pltpu.async_copy(idxt_hbm.at[pl.ds(j * nb + base, blk)],
                                 idx_v.at[s, j], sem_i)

        def wait_idx(s):
            for j in range(k):
                pltpu.make_async_copy(idxt_hbm.at[pl.ds(0, blk)],
                                      idx_v.at[s, j], sem_i).wait()

        def fire_gathers(s):
            for j in range(k):
                pltpu.async_copy(table_hbm.at[idx_v.at[s, j]],
                                 buf_v.at[s, j], sem_g)

        def drain_gathers(s):
            for j in range(k):
                pltpu.make_async_copy(table_hbm.at[idx_v.at[s, j]],
                                      buf_v.at[s, j], sem_g).wait()

        def start_out(b, s):
            base = (b * _NW + wid) * blk
            pltpu.async_copy(buf_v.at[s, 0], out_hbm.at[pl.ds(base, blk)],
                             sem_o)

        def wait_out(s):
            pltpu.make_async_copy(buf_v.at[s, 0], out_hbm.at[pl.ds(0, blk)],
                                  sem_o).wait()

        def reduce_block(s):
            if k == 1:
                return

            def row(r, c2):
                if bf:
                    fmt = plsc.PackFormat.INTERLEAVED
                    for j in range(d // 32):
                        sl = pl.ds(j * 32, 32)
                        acc_a, acc_b = plsc.unpack(buf_v[s, 0, r, sl],
                                                   format=fmt)
                        for q in range(1, k):
                            qa, qb = plsc.unpack(buf_v[s, q, r, sl],
                                                 format=fmt)
                            acc_a = acc_a + qa
                            acc_b = acc_b + qb
                        buf_v[s, 0, r, sl] = plsc.pack(acc_a, acc_b,
                                                       format=fmt)
                else:
                    for j in range(d // 16):
                        sl = pl.ds(j * 16, 16)
                        acc = buf_v[s, 0, r, sl]
                        for q in range(1, k):
                            acc = acc + buf_v[s, q, r, sl]
                        buf_v[s, 0, r, sl] = acc
                return c2

            lax.fori_loop(0, blk, row, 0)

        def step(b, s):
            drain_gathers(s)

            @pl.when(b + ns - 1 < bpw)
            def _():
                sg = (s + ns - 1) % ns
                wait_idx(sg)

                @pl.when(b >= 1)
                def _():
                    wait_out(sg)

                fire_gathers(sg)

                @pl.when(b + ns < bpw)
                def _():
                    start_idx(b + ns, s)

            reduce_block(s)
            start_out(b, s)

        for j in range(ns - 1):
            start_idx(j, j)
            wait_idx(j)
            fire_gathers(j)
        start_idx(ns - 1, ns - 1)

        def rotation(i, carry):
            for j in range(ns):
                step(ns * i + j, j)
            return carry

        lax.fori_loop(0, bpw // ns, rotation, 0)
        for j in range(ns):
            wait_out(j)

    dt = jnp.bfloat16 if bf else jnp.float32
    return pl.kernel(
        body,
        out_type=jax.ShapeDtypeStruct((nb, d), dt),
        mesh=mesh,
        compiler_params=pltpu.CompilerParams(
            use_tc_tiling_on_sc=False,
            needs_layout_passes=not bf),
        scratch_types=[
            pltpu.VMEM((ns, k, blk), jnp.int32),
            pltpu.VMEM((ns, k, blk, d), dt),
            pltpu.SemaphoreType.DMA,
            pltpu.SemaphoreType.DMA,
            pltpu.SemaphoreType.DMA,
        ],
    )


def _gsum(table, idxt):
    k, nb = idxt.shape
    bf = table.dtype == jnp.bfloat16
    return _make_gsum(nb, k, table.shape[1], bf)(table, idxt.reshape(-1))


def _dot_t(x, w):
    return lax.dot_general(x, w, (((1,), (1,)), ((), ())),
                           preferred_element_type=jnp.float32)


def _prologue_body(fb_ref, ms_ref, we_ref, w2_ref, h0_ref, c_ref):
    h0 = jnp.maximum(_dot_t(fb_ref[...], we_ref[...]), 0.0)
    h0_ref[...] = h0.astype(jnp.bfloat16)
    c_ref[...] = h0 + _dot_t(ms_ref[...], w2_ref[...])


def _iter_body(s_ref, c_ref, w1_ref, o_ref):
    h = jnp.maximum(c_ref[...] + _dot_t(s_ref[...], w1_ref[...]), 0.0)
    o_ref[...] = h.astype(jnp.bfloat16)


def _final_body(sa_ref, ms_ref, wo1_ref, wo2_ref, o_ref):
    acc = _dot_t(sa_ref[...], wo1_ref[...]) + _dot_t(ms_ref[...], wo2_ref[...])
    o_ref[...] = jnp.maximum(acc, 0.0)


def _full(shape):
    return pl.BlockSpec(shape, lambda i: (0, 0))


def _rows(bm, d):
    return pl.BlockSpec((bm, d), lambda i: (i, 0))


def kernel(fatoms, fbonds, aoutgraph, bgraph, aingraph, scope, all_bonds,
           W_ein, W_edge, W_eout):
    f32 = jnp.float32
    n_b, bf = fbonds.shape
    n_a = fatoms.shape[0]

    i32 = jnp.int32
    bf16 = jnp.bfloat16
    fat_z = jnp.pad(jnp.concatenate(
        [jnp.zeros((1, _AF), f32), fatoms], axis=0),
        ((0, 7), (0, _AFP - _AF))).astype(bf16)
    sidx = jnp.concatenate([
        jnp.zeros((1,), i32),
        all_bonds[1:, 0].astype(i32) + 1,
        jnp.zeros((_EP - n_b,), i32),
    ]).reshape(1, _EP)
    bgt = jnp.pad(bgraph.astype(i32), ((0, _EP - n_b), (0, 0))).T
    agt = jnp.pad(aingraph.astype(i32), ((0, _AP - n_a), (0, 0))).T

    fb16 = jnp.pad(fbonds, ((0, 0), (0, 16 - bf)))
    we16 = jnp.pad(W_ein, ((0, 0), (0, 16 - bf)))
    w1b = W_edge[:, :_H].astype(bf16)
    w2b = jnp.pad(W_edge[:, _H:], ((0, 0), (0, _AFP - _AF))).astype(bf16)
    wo1b = W_eout[:, :_H].astype(bf16)
    wo2b = jnp.pad(W_eout[:, _H:], ((0, 0), (0, _AFP - _AF))).astype(bf16)

    msg = _gsum(fat_z, sidx)
    msum_e = _gsum(msg, bgt)
    msum_a = _gsum(msg, agt)

    bm = 640
    grid_e = n_b // bm
    h0, c = pl.pallas_call(
        _prologue_body,
        grid=(grid_e,),
        in_specs=[_rows(bm, 16), _rows(bm, _AFP),
                  _full((_H, 16)), _full((_H, _AFP))],
        out_specs=[_rows(bm, _H), _rows(bm, _H)],
        out_shape=[jax.ShapeDtypeStruct((n_b, _H), bf16),
                   jax.ShapeDtypeStruct((n_b, _H), f32)],
    )(fb16, msum_e, we16, w2b)

    h_e = h0
    for _ in range(3):
        s = _gsum(h_e, bgt)
        h_e = pl.pallas_call(
            _iter_body,
            grid=(grid_e,),
            in_specs=[_rows(bm, _H), _rows(bm, _H), _full((_H, _H))],
            out_specs=_rows(bm, _H),
            out_shape=jax.ShapeDtypeStruct((n_b, _H), bf16),
        )(s, c, w1b)

    s_a = _gsum(h_e, agt)
    bma = 512
    out_p = pl.pallas_call(
        _final_body,
        grid=(_AP // bma,),
        in_specs=[_rows(bma, _H), _rows(bma, _AFP),
                  _full((_H, _H)), _full((_H, _AFP))],
        out_specs=_rows(bma, _H),
        out_shape=jax.ShapeDtypeStruct((_AP, _H), f32),
    )(s_a, msum_a, wo1b, wo2b)

    return out_p[:n_a].T
